# initial kernel scaffold (unmeasured)
import jax
import jax.numpy as jnp
from jax import lax
from jax.experimental import pallas as pl
from jax.experimental.pallas import tpu as pltpu

N_DEV = 16
CHUNK = 256

RING = [0, 4, 8, 12, 13, 9, 5, 1, 2, 6, 10, 14, 15, 11, 7, 3]
POS = [RING.index(p) for p in range(N_DEV)]
SUCC = [0] * N_DEV
PRED = [0] * N_DEV
for _a, _p in enumerate(RING):
    SUCC[_p] = RING[(_a + 1) % N_DEV]
    PRED[_p] = RING[(_a - 1) % N_DEV]


def _lut(idx, table):
    v = jnp.int32(table[0])
    for k in range(1, N_DEV):
        v = jnp.where(idx == k, jnp.int32(table[k]), v)
    return v


def kernel(x, w_mat, scale_x, scale_w):
    m, k_loc = x.shape
    _, n = w_mat.shape
    assert m == N_DEV * CHUNK

    def body(x_ref, w_ref, sx_ref, sw_ref, out_ref,
             wb_ref, cur_ref, recv_ref,
             send_sems, recv_sems, out_sems, own_sem, credit_sem):
        p = lax.axis_index("i")
        j = _lut(p, POS)
        succ = _lut(p, SUCC)
        pred = _lut(p, PRED)

        barrier = pltpu.get_barrier_semaphore()
        for nbr in (succ, pred):
            pl.semaphore_signal(barrier, inc=1, device_id=(nbr,),
                                device_id_type=pl.DeviceIdType.MESH)
        pl.semaphore_wait(barrier, 2)

        wb_ref[...] = w_ref[...].astype(jnp.bfloat16)

        def partial_chunk(c):
            xc = x_ref[pl.ds(c * CHUNK, CHUNK), :].astype(jnp.bfloat16)
            return jnp.dot(xc, wb_ref[...], preferred_element_type=jnp.float32)

        cur_ref[0] = partial_chunk(j)

        for t in range(N_DEV - 1):
            slot = t % 2
            rdma = pltpu.make_async_remote_copy(
                src_ref=cur_ref.at[slot],
                dst_ref=recv_ref.at[slot],
                send_sem=send_sems.at[slot],
                recv_sem=recv_sems.at[slot],
                device_id=(succ,),
                device_id_type=pl.DeviceIdType.MESH,
            )
            if t >= 2:
                pl.semaphore_wait(credit_sem, 1)
            rdma.start()
            pc = partial_chunk((j - (t + 1)) % N_DEV)
            rdma.wait()
            cur_ref[(t + 1) % 2] = pc + recv_ref[slot]
            pl.semaphore_signal(credit_sem, inc=1, device_id=(pred,),
                                device_id_type=pl.DeviceIdType.MESH)

        scale = sx_ref[0] * sw_ref[0]
        own = (j + 1) % N_DEV
        cur_ref[1] = jnp.maximum(cur_ref[1] * scale, 0.0)
        own_copy = pltpu.make_async_copy(
            cur_ref.at[1], out_ref.at[pl.ds(own * CHUNK, CHUNK), :], own_sem)
        own_copy.start()

        prev_copy = None
        for t in range(N_DEV - 1, 2 * N_DEV - 2):
            slot = t % 2
            if t == N_DEV - 1:
                src = cur_ref.at[1]
            else:
                src = recv_ref.at[(t + 1) % 2]
            rdma = pltpu.make_async_remote_copy(
                src_ref=src,
                dst_ref=recv_ref.at[slot],
                send_sem=send_sems.at[slot],
                recv_sem=recv_sems.at[slot],
                device_id=(succ,),
                device_id_type=pl.DeviceIdType.MESH,
            )
            pl.semaphore_wait(credit_sem, 1)
            rdma.start()
            rdma.wait()
            if prev_copy is not None:
                prev_copy.wait()
                if t <= 2 * N_DEV - 4:
                    pl.semaphore_signal(credit_sem, inc=1, device_id=(pred,),
                                        device_id_type=pl.DeviceIdType.MESH)
            rc = (j - (t - (N_DEV - 1))) % N_DEV
            prev_copy = pltpu.make_async_copy(
                recv_ref.at[slot],
                out_ref.at[pl.ds(rc * CHUNK, CHUNK), :],
                out_sems.at[slot])
            prev_copy.start()
        prev_copy.wait()
        own_copy.wait()

    return pl.pallas_call(
        body,
        out_shape=jax.ShapeDtypeStruct((m, n), jnp.float32),
        in_specs=[
            pl.BlockSpec(memory_space=pltpu.VMEM),
            pl.BlockSpec(memory_space=pltpu.VMEM),
            pl.BlockSpec(memory_space=pltpu.SMEM),
            pl.BlockSpec(memory_space=pltpu.SMEM),
        ],
        out_specs=pl.BlockSpec(memory_space=pltpu.ANY),
        scratch_shapes=[
            pltpu.VMEM((k_loc, n), jnp.bfloat16),
            pltpu.VMEM((2, CHUNK, n), jnp.float32),
            pltpu.VMEM((2, CHUNK, n), jnp.float32),
            pltpu.SemaphoreType.DMA((2,)),
            pltpu.SemaphoreType.DMA((2,)),
            pltpu.SemaphoreType.DMA((2,)),
            pltpu.SemaphoreType.DMA,
            pltpu.SemaphoreType.REGULAR,
        ],
        compiler_params=pltpu.CompilerParams(
            collective_id=0,
            vmem_limit_bytes=110 * 1024 * 1024,
        ),
    )(x, w_mat, scale_x, scale_w)


# baseline (device time: 2876457 ns/iter reference)
import jax
import jax.numpy as jnp
from jax import lax
from jax.experimental import pallas as pl
from jax.experimental.pallas import tpu as pltpu

N_DEV = 16
CHUNK = 256

RING = [0, 4, 8, 12, 13, 9, 5, 1, 2, 6, 10, 14, 15, 11, 7, 3]
POS = [RING.index(p) for p in range(N_DEV)]
SUCC = [0] * N_DEV
PRED = [0] * N_DEV
for _a, _p in enumerate(RING):
    SUCC[_p] = RING[(_a + 1) % N_DEV]
    PRED[_p] = RING[(_a - 1) % N_DEV]


def _lut(idx, table):
    v = jnp.int32(table[0])
    for k in range(1, N_DEV):
        v = jnp.where(idx == k, jnp.int32(table[k]), v)
    return v


def kernel(x, w_mat, scale_x, scale_w):
    m, k_loc = x.shape
    _, n = w_mat.shape
    assert m == N_DEV * CHUNK

    def body(x_ref, w_ref, sx_ref, sw_ref, out_ref,
             wb_ref, cur_ref, recv_ref,
             send_sems, recv_sems, out_sems, own_sem, credit_sem):
        p = lax.axis_index("i")
        j = _lut(p, POS)
        succ = _lut(p, SUCC)
        pred = _lut(p, PRED)

        barrier = pltpu.get_barrier_semaphore()
        for nbr in (succ, pred):
            pl.semaphore_signal(barrier, inc=1, device_id=(nbr,),
                                device_id_type=pl.DeviceIdType.MESH)
        pl.semaphore_wait(barrier, 2)

        wb_ref[...] = w_ref[...].astype(jnp.bfloat16)

        def partial_chunk(c):
            xc = x_ref[pl.ds(c * CHUNK, CHUNK), :].astype(jnp.bfloat16)
            return jnp.dot(xc, wb_ref[...], preferred_element_type=jnp.float32)

        cur_ref[0] = partial_chunk(j)

        for t in range(N_DEV - 1):
            slot = t % 2
            rdma = pltpu.make_async_remote_copy(
                src_ref=cur_ref.at[slot],
                dst_ref=recv_ref.at[slot],
                send_sem=send_sems.at[slot],
                recv_sem=recv_sems.at[slot],
                device_id=(succ,),
                device_id_type=pl.DeviceIdType.MESH,
            )
            if t >= 2:
                pl.semaphore_wait(credit_sem, 1)
            rdma.start()
            pc = partial_chunk((j - (t + 1)) % N_DEV)
            rdma.wait()
            cur_ref[(t + 1) % 2] = pc + recv_ref[slot]
            pl.semaphore_signal(credit_sem, inc=1, device_id=(pred,),
                                device_id_type=pl.DeviceIdType.MESH)

        scale = sx_ref[0] * sw_ref[0]
        own = (j + 1) % N_DEV
        cur_ref[1] = jnp.maximum(cur_ref[1] * scale, 0.0)
        own_copy = pltpu.make_async_copy(
            cur_ref.at[1], out_ref.at[pl.ds(own * CHUNK, CHUNK), :], own_sem)
        own_copy.start()

        prev_copy = None
        for t in range(N_DEV - 1, 2 * N_DEV - 2):
            slot = t % 2
            if t == N_DEV - 1:
                src = cur_ref.at[1]
            else:
                src = recv_ref.at[(t + 1) % 2]
            rdma = pltpu.make_async_remote_copy(
                src_ref=src,
                dst_ref=recv_ref.at[slot],
                send_sem=send_sems.at[slot],
                recv_sem=recv_sems.at[slot],
                device_id=(succ,),
                device_id_type=pl.DeviceIdType.MESH,
            )
            pl.semaphore_wait(credit_sem, 1)
            rdma.start()
            rdma.wait()
            if prev_copy is not None:
                prev_copy.wait()
                if t <= 2 * N_DEV - 4:
                    pl.semaphore_signal(credit_sem, inc=1, device_id=(pred,),
                                        device_id_type=pl.DeviceIdType.MESH)
            rc = (j - (t - (N_DEV - 1))) % N_DEV
            prev_copy = pltpu.make_async_copy(
                recv_ref.at[slot],
                out_ref.at[pl.ds(rc * CHUNK, CHUNK), :],
                out_sems.at[slot])
            prev_copy.start()
        prev_copy.wait()
        own_copy.wait()

    return pl.pallas_call(
        body,
        out_shape=jax.ShapeDtypeStruct((m, n), jnp.float32),
        in_specs=[
            pl.BlockSpec(memory_space=pltpu.VMEM),
            pl.BlockSpec(memory_space=pltpu.VMEM),
            pl.BlockSpec(memory_space=pltpu.SMEM),
            pl.BlockSpec(memory_space=pltpu.SMEM),
        ],
        out_specs=pl.BlockSpec(memory_space=pl.ANY),
        scratch_shapes=[
            pltpu.VMEM((k_loc, n), jnp.bfloat16),
            pltpu.VMEM((2, CHUNK, n), jnp.float32),
            pltpu.VMEM((2, CHUNK, n), jnp.float32),
            pltpu.SemaphoreType.DMA((2,)),
            pltpu.SemaphoreType.DMA((2,)),
            pltpu.SemaphoreType.DMA((2,)),
            pltpu.SemaphoreType.DMA,
            pltpu.SemaphoreType.REGULAR,
        ],
        compiler_params=pltpu.CompilerParams(
            collective_id=0,
            vmem_limit_bytes=110 * 1024 * 1024,
        ),
    )(x, w_mat, scale_x, scale_w)


# device time: 860399 ns/iter; 3.3432x vs baseline; 3.3432x over previous
import jax
import jax.numpy as jnp
from jax import lax
from jax.experimental import pallas as pl
from jax.experimental.pallas import tpu as pltpu

N_DEV = 16
CHUNK = 256

RING = [0, 4, 8, 12, 13, 9, 5, 1, 2, 6, 10, 14, 15, 11, 7, 3]
POS = [RING.index(p) for p in range(N_DEV)]
SUCC = [0] * N_DEV
PRED = [0] * N_DEV
for _a, _p in enumerate(RING):
    SUCC[_p] = RING[(_a + 1) % N_DEV]
    PRED[_p] = RING[(_a - 1) % N_DEV]


def _lut(idx, table):
    v = jnp.int32(table[0])
    for k in range(1, N_DEV):
        v = jnp.where(idx == k, jnp.int32(table[k]), v)
    return v


class _Dir:

    def __init__(self, to_id, from_id, col0, send, recv, stage, own,
                 send_sems, recv_sems, out_sems, own_sem, credit):
        self.to_id = to_id
        self.from_id = from_id
        self.col0 = col0
        self.send = send
        self.recv = recv
        self.stage = stage
        self.own = own
        self.send_sems = send_sems
        self.recv_sems = recv_sems
        self.out_sems = out_sems
        self.own_sem = own_sem
        self.credit = credit
        self.copies = {}


def kernel(x, w_mat, scale_x, scale_w):
    m, k_loc = x.shape
    _, n = w_mat.shape
    n2 = n // 2
    assert m == N_DEV * CHUNK

    def body(x_ref, w_ref, sx_ref, sw_ref, out_ref,
             wb_ref,
             sendA, recvA, stageA, ownA,
             sendB, recvB, stageB, ownB,
             send_semsA, recv_semsA, out_semsA, own_semA, creditA,
             send_semsB, recv_semsB, out_semsB, own_semB, creditB):
        p = lax.axis_index("i")
        j = _lut(p, POS)
        succ = _lut(p, SUCC)
        pred = _lut(p, PRED)

        A = _Dir(succ, pred, 0, sendA, recvA, stageA, ownA,
                 send_semsA, recv_semsA, out_semsA, own_semA, creditA)
        B = _Dir(pred, succ, n2, sendB, recvB, stageB, ownB,
                 send_semsB, recv_semsB, out_semsB, own_semB, creditB)
        dirs = (A, B)

        barrier = pltpu.get_barrier_semaphore()
        for nbr in (succ, pred):
            pl.semaphore_signal(barrier, inc=1, device_id=(nbr,),
                                device_id_type=pl.DeviceIdType.MESH)
        pl.semaphore_wait(barrier, 2)

        wb_ref[...] = w_ref[...].astype(jnp.bfloat16)

        def partial(c, col0):
            xc = x_ref[pl.ds(c * CHUNK, CHUNK), :].astype(jnp.bfloat16)
            return jnp.dot(xc, wb_ref[:, pl.ds(col0, n2)],
                           preferred_element_type=jnp.float32)

        A.send[0] = partial(j, A.col0).astype(jnp.bfloat16)
        B.send[0] = partial(j, B.col0).astype(jnp.bfloat16)

        scale = sx_ref[0] * sw_ref[0]

        for t in range(N_DEV - 1):
            slot = t % 2
            rdmas = []
            for d in dirs:
                if t >= 2:
                    pl.semaphore_wait(d.credit, 1)
                r = pltpu.make_async_remote_copy(
                    src_ref=d.send.at[slot],
                    dst_ref=d.recv.at[slot],
                    send_sem=d.send_sems.at[slot],
                    recv_sem=d.recv_sems.at[slot],
                    device_id=(d.to_id,),
                    device_id_type=pl.DeviceIdType.MESH,
                )
                r.start()
                rdmas.append(r)
            pcA = partial((j - (t + 1)) % N_DEV, A.col0)
            pcB = partial((j + (t + 1)) % N_DEV, B.col0)
            for d, r, pc in zip(dirs, rdmas, (pcA, pcB)):
                r.wait()
                acc = pc + d.recv[slot].astype(jnp.float32)
                if t < N_DEV - 2:
                    d.send[(t + 1) % 2] = acc.astype(jnp.bfloat16)
                else:
                    fin = jnp.maximum(acc * scale, 0.0)
                    d.own[...] = fin
                    d.send[1] = fin.astype(jnp.bfloat16)
                pl.semaphore_signal(d.credit, inc=1, device_id=(d.from_id,),
                                    device_id_type=pl.DeviceIdType.MESH)

        ownA_c = (j + 1) % N_DEV
        ownB_c = (j - 1) % N_DEV
        own_copies = []
        for d, oc in ((A, ownA_c), (B, ownB_c)):
            cp = pltpu.make_async_copy(
                d.own,
                out_ref.at[pl.ds(oc * CHUNK, CHUNK), pl.ds(d.col0, n2)],
                d.own_sem)
            cp.start()
            own_copies.append(cp)

        for t in range(N_DEV - 1, 2 * N_DEV - 2):
            slot = t % 2
            rdmas = []
            for d in dirs:
                if t == N_DEV - 1:
                    src = d.send.at[1]
                else:
                    src = d.recv.at[(t + 1) % 2]
                pl.semaphore_wait(d.credit, 1)
                r = pltpu.make_async_remote_copy(
                    src_ref=src,
                    dst_ref=d.recv.at[slot],
                    send_sem=d.send_sems.at[slot],
                    recv_sem=d.recv_sems.at[slot],
                    device_id=(d.to_id,),
                    device_id_type=pl.DeviceIdType.MESH,
                )
                r.start()
                rdmas.append(r)
            rcA = (j - (t - (N_DEV - 1))) % N_DEV
            rcB = (j + (t - (N_DEV - 1))) % N_DEV
            for d, r, rc in zip(dirs, rdmas, (rcA, rcB)):
                r.wait()
                if t >= N_DEV:
                    if t <= 2 * N_DEV - 4:
                        pl.semaphore_signal(
                            d.credit, inc=1, device_id=(d.from_id,),
                            device_id_type=pl.DeviceIdType.MESH)
                if t - 2 in d.copies:
                    d.copies[t - 2].wait()
                d.stage[slot] = d.recv[slot].astype(jnp.float32)
                cp = pltpu.make_async_copy(
                    d.stage.at[slot],
                    out_ref.at[pl.ds(rc * CHUNK, CHUNK), pl.ds(d.col0, n2)],
                    d.out_sems.at[slot])
                cp.start()
                d.copies[t] = cp

        for d in dirs:
            d.copies[2 * N_DEV - 4].wait()
            d.copies[2 * N_DEV - 3].wait()
        for cp in own_copies:
            cp.wait()

    return pl.pallas_call(
        body,
        out_shape=jax.ShapeDtypeStruct((m, n), jnp.float32),
        in_specs=[
            pl.BlockSpec(memory_space=pltpu.VMEM),
            pl.BlockSpec(memory_space=pltpu.VMEM),
            pl.BlockSpec(memory_space=pltpu.SMEM),
            pl.BlockSpec(memory_space=pltpu.SMEM),
        ],
        out_specs=pl.BlockSpec(memory_space=pl.ANY),
        scratch_shapes=[
            pltpu.VMEM((k_loc, n), jnp.bfloat16),
            pltpu.VMEM((2, CHUNK, n2), jnp.bfloat16),
            pltpu.VMEM((2, CHUNK, n2), jnp.bfloat16),
            pltpu.VMEM((2, CHUNK, n2), jnp.float32),
            pltpu.VMEM((CHUNK, n2), jnp.float32),
            pltpu.VMEM((2, CHUNK, n2), jnp.bfloat16),
            pltpu.VMEM((2, CHUNK, n2), jnp.bfloat16),
            pltpu.VMEM((2, CHUNK, n2), jnp.float32),
            pltpu.VMEM((CHUNK, n2), jnp.float32),
            pltpu.SemaphoreType.DMA((2,)),
            pltpu.SemaphoreType.DMA((2,)),
            pltpu.SemaphoreType.DMA((2,)),
            pltpu.SemaphoreType.DMA,
            pltpu.SemaphoreType.REGULAR,
            pltpu.SemaphoreType.DMA((2,)),
            pltpu.SemaphoreType.DMA((2,)),
            pltpu.SemaphoreType.DMA((2,)),
            pltpu.SemaphoreType.DMA,
            pltpu.SemaphoreType.REGULAR,
        ],
        compiler_params=pltpu.CompilerParams(
            collective_id=0,
            vmem_limit_bytes=110 * 1024 * 1024,
        ),
    )(x, w_mat, scale_x, scale_w)


# device time: 860384 ns/iter; 3.3432x vs baseline; 1.0000x over previous
import jax
import jax.numpy as jnp
from jax import lax
from jax.experimental import pallas as pl
from jax.experimental.pallas import tpu as pltpu

N_DEV = 16
CHUNK = 256

RING = [0, 4, 8, 12, 13, 9, 5, 1, 2, 6, 10, 14, 15, 11, 7, 3]
POS = [RING.index(p) for p in range(N_DEV)]
SUCC = [0] * N_DEV
PRED = [0] * N_DEV
for _a, _p in enumerate(RING):
    SUCC[_p] = RING[(_a + 1) % N_DEV]
    PRED[_p] = RING[(_a - 1) % N_DEV]


def _lut(idx, table):
    v = jnp.int32(table[0])
    for k in range(1, N_DEV):
        v = jnp.where(idx == k, jnp.int32(table[k]), v)
    return v


class _Dir:

    def __init__(self, to_id, from_id, col0, send, recv, stage, own,
                 send_sems, recv_sems, out_sems, own_sem, credit):
        self.to_id = to_id
        self.from_id = from_id
        self.col0 = col0
        self.send = send
        self.recv = recv
        self.stage = stage
        self.own = own
        self.send_sems = send_sems
        self.recv_sems = recv_sems
        self.out_sems = out_sems
        self.own_sem = own_sem
        self.credit = credit
        self.copies = {}


def kernel(x, w_mat, scale_x, scale_w):
    m, k_loc = x.shape
    _, n = w_mat.shape
    n2 = n // 2
    assert m == N_DEV * CHUNK

    def body(x_ref, w_ref, sx_ref, sw_ref, out_ref,
             wb_ref,
             sendA, recvA, stageA, ownA,
             sendB, recvB, stageB, ownB,
             send_semsA, recv_semsA, out_semsA, own_semA, creditA,
             send_semsB, recv_semsB, out_semsB, own_semB, creditB):
        p = lax.axis_index("i")
        j = _lut(p, POS)
        succ = _lut(p, SUCC)
        pred = _lut(p, PRED)

        A = _Dir(succ, pred, 0, sendA, recvA, stageA, ownA,
                 send_semsA, recv_semsA, out_semsA, own_semA, creditA)
        B = _Dir(pred, succ, n2, sendB, recvB, stageB, ownB,
                 send_semsB, recv_semsB, out_semsB, own_semB, creditB)
        dirs = (A, B)

        barrier = pltpu.get_barrier_semaphore()
        for nbr in (succ, pred):
            pl.semaphore_signal(barrier, inc=1, device_id=(nbr,),
                                device_id_type=pl.DeviceIdType.MESH)
        pl.semaphore_wait(barrier, 2)

        wb_ref[...] = w_ref[...].astype(jnp.bfloat16)

        def partial(c, col0):
            xc = x_ref[pl.ds(c * CHUNK, CHUNK), :].astype(jnp.bfloat16)
            return jnp.dot(xc, wb_ref[:, pl.ds(col0, n2)],
                           preferred_element_type=jnp.float32)

        A.send[0] = partial(j, A.col0).astype(jnp.bfloat16)
        B.send[0] = partial(j, B.col0).astype(jnp.bfloat16)

        scale = sx_ref[0] * sw_ref[0]

        prev_rdmas = {}
        for t in range(N_DEV - 1):
            slot = t % 2
            rdmas = []
            for d in dirs:
                if t >= 2:
                    pl.semaphore_wait(d.credit, 1)
                r = pltpu.make_async_remote_copy(
                    src_ref=d.send.at[slot],
                    dst_ref=d.recv.at[slot],
                    send_sem=d.send_sems.at[slot],
                    recv_sem=d.recv_sems.at[slot],
                    device_id=(d.to_id,),
                    device_id_type=pl.DeviceIdType.MESH,
                )
                r.start()
                rdmas.append(r)
            pcA = partial((j - (t + 1)) % N_DEV, A.col0)
            pcB = partial((j + (t + 1)) % N_DEV, B.col0)
            for i, (d, r, pc) in enumerate(zip(dirs, rdmas, (pcA, pcB))):
                r.wait_recv()
                if t >= 1:
                    prev_rdmas[i].wait_send()
                acc = pc + d.recv[slot].astype(jnp.float32)
                if t < N_DEV - 2:
                    d.send[(t + 1) % 2] = acc.astype(jnp.bfloat16)
                else:
                    fin = jnp.maximum(acc * scale, 0.0)
                    d.own[...] = fin
                    d.send[1] = fin.astype(jnp.bfloat16)
                pl.semaphore_signal(d.credit, inc=1, device_id=(d.from_id,),
                                    device_id_type=pl.DeviceIdType.MESH)
                prev_rdmas[i] = r

        ownA_c = (j + 1) % N_DEV
        ownB_c = (j - 1) % N_DEV
        own_copies = []
        for d, oc in ((A, ownA_c), (B, ownB_c)):
            cp = pltpu.make_async_copy(
                d.own,
                out_ref.at[pl.ds(oc * CHUNK, CHUNK), pl.ds(d.col0, n2)],
                d.own_sem)
            cp.start()
            own_copies.append(cp)

        for r in prev_rdmas.values():
            r.wait_send()

        def convert_and_store(d, s):
            pslot = s % 2
            if (s - 2) in d.copies:
                d.copies[s - 2].wait()
            d.stage[pslot] = d.recv[pslot].astype(jnp.float32)
            if d is A:
                rc = (j - (s - (N_DEV - 1))) % N_DEV
            else:
                rc = (j + (s - (N_DEV - 1))) % N_DEV
            cp = pltpu.make_async_copy(
                d.stage.at[pslot],
                out_ref.at[pl.ds(rc * CHUNK, CHUNK), pl.ds(d.col0, n2)],
                d.out_sems.at[pslot])
            cp.start()
            d.copies[s] = cp

        for t in range(N_DEV - 1, 2 * N_DEV - 2):
            slot = t % 2
            rdmas = []
            for d in dirs:
                pl.semaphore_wait(d.credit, 1)
                if t == N_DEV - 1:
                    src = d.send.at[1]
                else:
                    src = d.recv.at[(t + 1) % 2]
                r = pltpu.make_async_remote_copy(
                    src_ref=src,
                    dst_ref=d.recv.at[slot],
                    send_sem=d.send_sems.at[slot],
                    recv_sem=d.recv_sems.at[slot],
                    device_id=(d.to_id,),
                    device_id_type=pl.DeviceIdType.MESH,
                )
                r.start()
                rdmas.append(r)
            if t >= N_DEV:
                for d in dirs:
                    convert_and_store(d, t - 1)
            for d, r in zip(dirs, rdmas):
                r.wait_recv()
            for d, r in zip(dirs, rdmas):
                r.wait_send()
                if N_DEV <= t <= 2 * N_DEV - 4:
                    pl.semaphore_signal(
                        d.credit, inc=1, device_id=(d.from_id,),
                        device_id_type=pl.DeviceIdType.MESH)

        for d in dirs:
            convert_and_store(d, 2 * N_DEV - 3)
            d.copies[2 * N_DEV - 4].wait()
            d.copies[2 * N_DEV - 3].wait()
        for cp in own_copies:
            cp.wait()

    return pl.pallas_call(
        body,
        out_shape=jax.ShapeDtypeStruct((m, n), jnp.float32),
        in_specs=[
            pl.BlockSpec(memory_space=pltpu.VMEM),
            pl.BlockSpec(memory_space=pltpu.VMEM),
            pl.BlockSpec(memory_space=pltpu.SMEM),
            pl.BlockSpec(memory_space=pltpu.SMEM),
        ],
        out_specs=pl.BlockSpec(memory_space=pl.ANY),
        scratch_shapes=[
            pltpu.VMEM((k_loc, n), jnp.bfloat16),
            pltpu.VMEM((2, CHUNK, n2), jnp.bfloat16),
            pltpu.VMEM((2, CHUNK, n2), jnp.bfloat16),
            pltpu.VMEM((2, CHUNK, n2), jnp.float32),
            pltpu.VMEM((CHUNK, n2), jnp.float32),
            pltpu.VMEM((2, CHUNK, n2), jnp.bfloat16),
            pltpu.VMEM((2, CHUNK, n2), jnp.bfloat16),
            pltpu.VMEM((2, CHUNK, n2), jnp.float32),
            pltpu.VMEM((CHUNK, n2), jnp.float32),
            pltpu.SemaphoreType.DMA((2,)),
            pltpu.SemaphoreType.DMA((2,)),
            pltpu.SemaphoreType.DMA((2,)),
            pltpu.SemaphoreType.DMA,
            pltpu.SemaphoreType.REGULAR,
            pltpu.SemaphoreType.DMA((2,)),
            pltpu.SemaphoreType.DMA((2,)),
            pltpu.SemaphoreType.DMA((2,)),
            pltpu.SemaphoreType.DMA,
            pltpu.SemaphoreType.REGULAR,
        ],
        compiler_params=pltpu.CompilerParams(
            collective_id=0,
            vmem_limit_bytes=110 * 1024 * 1024,
        ),
    )(x, w_mat, scale_x, scale_w)
